# baseline (speedup 1.0000x reference)
import jax
import jax.numpy as jnp
from jax.experimental import pallas as pl
from jax.experimental.pallas import tpu as pltpu


def _body(I_ref, d_ref, zl_ref, o_zl_ref, o_zu_ref):
    m = I_ref[...].astype(jnp.float32)
    dI = d_ref[...] * m
    zlI = zl_ref[...] * m
    o_zl_ref[...] = zlI * jnp.maximum(-dI, 0.0)
    o_zu_ref[...] = -(zlI * jnp.maximum(dI, 0.0))


def kernel(I, d, zl):
    B, n = d.shape
    out = jax.ShapeDtypeStruct((B, n), jnp.float32)
    return pl.pallas_call(
        _body,
        out_shape=(out, out),
        in_specs=[
            pl.BlockSpec(memory_space=pltpu.VMEM),
            pl.BlockSpec(memory_space=pltpu.VMEM),
            pl.BlockSpec(memory_space=pltpu.VMEM),
        ],
        out_specs=(
            pl.BlockSpec(memory_space=pltpu.VMEM),
            pl.BlockSpec(memory_space=pltpu.VMEM),
        ),
    )(I, d, zl)
